# baseline (speedup 1.0000x reference)
import jax
import jax.numpy as jnp
from jax.experimental import pallas as pl
from jax.experimental.pallas import tpu as pltpu

_NBUF = 6
_CH = 80


def _gcn_kernel(x_ref, w_ref, bias_ref, adj_hbm, out_ref, support_ref,
                bufs_ref, sems):
    i = pl.program_id(0)
    cpb = out_ref.shape[0] // _CH
    nchunks = pl.num_programs(0) * cpb

    def start_copy(g, slot):
        pltpu.make_async_copy(
            adj_hbm.at[pl.ds(g * _CH, _CH), :],
            bufs_ref.at[slot],
            sems.at[slot],
        ).start()

    @pl.when(i == 0)
    def _():
        support_ref[...] = jnp.dot(
            x_ref[...], w_ref[...], preferred_element_type=jnp.float32
        )
        for s in range(_NBUF):
            start_copy(s, s)

    def body(j, carry):
        g = i * cpb + j
        slot = g % _NBUF
        pltpu.make_async_copy(
            adj_hbm.at[pl.ds(g * _CH, _CH), :],
            bufs_ref.at[slot],
            sems.at[slot],
        ).wait()
        out_ref[pl.ds(j * _CH, _CH), :] = (
            jnp.dot(bufs_ref[slot], support_ref[...],
                    preferred_element_type=jnp.float32)
            + bias_ref[...]
        )
        nxt = g + _NBUF

        @pl.when(nxt < nchunks)
        def _():
            start_copy(nxt, slot)

        return carry

    jax.lax.fori_loop(0, cpb, body, 0)


def kernel(input, adj, weight, bias):
    n, din = input.shape
    dout = weight.shape[1]
    bm = next(b for b in (400, 80, 8, n) if n % b == 0 and b % _CH == 0)

    out = pl.pallas_call(
        _gcn_kernel,
        grid=(n // bm,),
        compiler_params=pltpu.CompilerParams(
            dimension_semantics=("arbitrary",),
        ),
        in_specs=[
            pl.BlockSpec((n, din), lambda i: (0, 0)),
            pl.BlockSpec((din, dout), lambda i: (0, 0)),
            pl.BlockSpec((1, dout), lambda i: (0, 0)),
            pl.BlockSpec(memory_space=pltpu.MemorySpace.HBM),
        ],
        out_specs=pl.BlockSpec((bm, dout), lambda i: (i, 0)),
        out_shape=jax.ShapeDtypeStruct((n, dout), jnp.float32),
        scratch_shapes=[
            pltpu.VMEM((n, dout), jnp.float32),
            pltpu.VMEM((_NBUF, _CH, n), jnp.float32),
            pltpu.SemaphoreType.DMA((_NBUF,)),
        ],
    )(input, weight, bias.reshape(1, dout), adj)
    return out


# speedup vs baseline: 1.0147x; 1.0147x over previous
import jax
import jax.numpy as jnp
from jax.experimental import pallas as pl
from jax.experimental.pallas import tpu as pltpu

_NBUF = 4
_CH = 80


def _gcn_kernel(x_ref, w_ref, bias_ref, adj_hbm, out_ref, support_ref,
                bufs_ref, sems):
    i = pl.program_id(0)
    cpb = out_ref.shape[0] // _CH
    nchunks = pl.num_programs(0) * cpb

    def start_copy(g, slot):
        pltpu.make_async_copy(
            adj_hbm.at[pl.ds(g * _CH, _CH), :],
            bufs_ref.at[slot],
            sems.at[slot],
        ).start()

    @pl.when(i == 0)
    def _():
        support_ref[...] = jnp.dot(
            x_ref[...], w_ref[...], preferred_element_type=jnp.float32
        )
        for s in range(_NBUF):
            start_copy(s, s)

    def body(j, carry):
        g = i * cpb + j
        slot = g % _NBUF
        pltpu.make_async_copy(
            adj_hbm.at[pl.ds(g * _CH, _CH), :],
            bufs_ref.at[slot],
            sems.at[slot],
        ).wait()
        out_ref[pl.ds(j * _CH, _CH), :] = (
            jnp.dot(bufs_ref[slot], support_ref[...],
                    preferred_element_type=jnp.float32)
            + bias_ref[...]
        )
        nxt = g + _NBUF

        @pl.when(nxt < nchunks)
        def _():
            start_copy(nxt, slot)

        return carry

    jax.lax.fori_loop(0, cpb, body, 0)


def kernel(input, adj, weight, bias):
    n, din = input.shape
    dout = weight.shape[1]
    bm = next(b for b in (400, 80, 8, n) if n % b == 0 and b % _CH == 0)

    out = pl.pallas_call(
        _gcn_kernel,
        grid=(n // bm,),
        compiler_params=pltpu.CompilerParams(
            dimension_semantics=("arbitrary",),
        ),
        in_specs=[
            pl.BlockSpec((n, din), lambda i: (0, 0)),
            pl.BlockSpec((din, dout), lambda i: (0, 0)),
            pl.BlockSpec((1, dout), lambda i: (0, 0)),
            pl.BlockSpec(memory_space=pltpu.MemorySpace.HBM),
        ],
        out_specs=pl.BlockSpec((bm, dout), lambda i: (i, 0)),
        out_shape=jax.ShapeDtypeStruct((n, dout), jnp.float32),
        scratch_shapes=[
            pltpu.VMEM((n, dout), jnp.float32),
            pltpu.VMEM((_NBUF, _CH, n), jnp.float32),
            pltpu.SemaphoreType.DMA((_NBUF,)),
        ],
    )(input, weight, bias.reshape(1, dout), adj)
    return out
